# baseline (speedup 1.0000x reference)
import functools

import jax
import jax.numpy as jnp
from jax import lax
from jax.experimental import pallas as pl
from jax.experimental.pallas import tpu as pltpu
from jax.experimental.pallas import tpu_sc as plsc

NC = 2
NS = 16
NW = NC * NS
GCH = 40
RB = 10
LAG = 5
SCH = 125
VB = 1250
EBLK = 1280


def _proj_body(x_ref, w_ref, o_ref):
    o_ref[0] = jnp.dot(x_ref[...], w_ref[0],
                       preferred_element_type=jnp.float32
                       ).astype(jnp.bfloat16)


def _proj(x, wstack):
    n, df = x.shape
    eh = wstack.shape[-1]
    return pl.pallas_call(
        _proj_body,
        grid=(4,),
        in_specs=[
            pl.BlockSpec((n, df), lambda i: (0, 0)),
            pl.BlockSpec((1, df, eh), lambda i: (i, 0, 0)),
        ],
        out_specs=pl.BlockSpec((1, n, eh), lambda i: (i, 0, 0)),
        out_shape=jax.ShapeDtypeStruct((4, n, eh), jnp.bfloat16),
    )(x, wstack)


def _make_gather(n, eh, h, ew):
    ngc = ew // GCH
    assert ngc % RB == 0 and RB == 2 * LAG
    nrounds = ngc // RB
    mesh = plsc.VectorSubcoreMesh(core_axis_name="c", subcore_axis_name="s")

    @functools.partial(
        pl.kernel,
        out_type=[jax.ShapeDtypeStruct((2, h, 2 * eh), jnp.bfloat16),
                  jax.ShapeDtypeStruct((2, h, 2 * eh), jnp.bfloat16)],
        mesh=mesh,
        compiler_params=pltpu.CompilerParams(use_tc_tiling_on_sc=False),
        scratch_types=[
            pltpu.VMEM((ew,), jnp.int32),
            pltpu.VMEM((ew,), jnp.int32),
            pltpu.VMEM((RB, GCH, eh), jnp.bfloat16),
            pltpu.VMEM((RB, GCH, eh), jnp.bfloat16),
            pltpu.SemaphoreType.DMA,
            pltpu.SemaphoreType.DMA,
        ],
    )
    def gather_k(pa0, pb0, pa1, pb1, ei0, ei1, ga, gb,
                 sidx, didx, bufa, bufb, gsem, ssem):
        cid = lax.axis_index("c")
        sid = lax.axis_index("s")
        wid = sid * NC + cid
        half = wid // NS
        rb0 = (wid % NS) * ew
        col0 = half * eh
        for t in range(2):
            pa = (pa0, pa1)[t]
            pb = (pb0, pb1)[t]
            ei = (ei0, ei1)[t]
            pltpu.sync_copy(ei.at[0].at[pl.ds(wid * ew, ew)], sidx)
            pltpu.sync_copy(ei.at[1].at[pl.ds(wid * ew, ew)], didx)
            for b in range(LAG):
                pltpu.async_copy(
                    pa.at[sidx.at[pl.ds(b * GCH, GCH)]], bufa.at[b], gsem)
                pltpu.async_copy(
                    pb.at[didx.at[pl.ds(b * GCH, GCH)]], bufb.at[b], gsem)

            def rnd(r, _, pa=pa, pb=pb, t=t):
                for b in range(RB):
                    c = r * RB + b
                    bn = (b + LAG) % RB
                    pltpu.make_async_copy(
                        pa.at[pl.ds(0, GCH)], bufa.at[b], gsem).wait()
                    pltpu.make_async_copy(
                        pb.at[pl.ds(0, GCH)], bufb.at[b], gsem).wait()
                    rb = rb0 + c * GCH
                    pltpu.async_copy(
                        bufa.at[b],
                        ga.at[t, pl.ds(rb, GCH), pl.ds(col0, eh)], ssem)
                    pltpu.async_copy(
                        bufb.at[b],
                        gb.at[t, pl.ds(rb, GCH), pl.ds(col0, eh)], ssem)

                    @pl.when(c >= LAG)
                    def _():
                        pltpu.make_async_copy(
                            bufa.at[bn],
                            ga.at[t, pl.ds(rb0, GCH), pl.ds(col0, eh)],
                            ssem).wait()
                        pltpu.make_async_copy(
                            bufb.at[bn],
                            gb.at[t, pl.ds(rb0, GCH), pl.ds(col0, eh)],
                            ssem).wait()

                    @pl.when(c + LAG < ngc)
                    def _():
                        pltpu.async_copy(
                            pa.at[sidx.at[pl.ds((c + LAG) * GCH, GCH)]],
                            bufa.at[bn], gsem)
                        pltpu.async_copy(
                            pb.at[didx.at[pl.ds((c + LAG) * GCH, GCH)]],
                            bufb.at[bn], gsem)
                return 0

            lax.fori_loop(0, nrounds, rnd, 0)
            for _ in range(LAG):
                pltpu.make_async_copy(
                    bufa.at[0],
                    ga.at[t, pl.ds(rb0, GCH), pl.ds(col0, eh)], ssem).wait()
                pltpu.make_async_copy(
                    bufb.at[0],
                    gb.at[t, pl.ds(rb0, GCH), pl.ds(col0, eh)], ssem).wait()

    return gather_k


def _edge_mlp_body(eh, de, ga_ref, gb_ref, eal_ref, ear_ref, w1c_ref,
                   b1_ref, w2_ref, b2_ref, o_ref):
    g2 = ga_ref[0].astype(jnp.float32) + gb_ref[0].astype(jnp.float32)
    for side, ea_ref in ((0, eal_ref), (1, ear_ref)):
        g = g2[:, side * eh:(side + 1) * eh]
        contrib = lax.dot_general(ea_ref[...], w1c_ref[...],
                                  (((0,), (0,)), ((), ())),
                                  preferred_element_type=jnp.float32)
        pre = g + contrib + b1_ref[...]
        hid = jnp.maximum(pre, 0.0)
        o_ref[side] = jnp.dot(hid, w2_ref[...],
                              preferred_element_type=jnp.float32) \
            + b2_ref[...]


def _edge_mlp(t, gaf, gbf, eaT, w1c, b1, w2, b2, h):
    nblk = h // EBLK
    eh = w1c.shape[-1]
    de = eaT.shape[0]
    return pl.pallas_call(
        functools.partial(_edge_mlp_body, eh, de),
        grid=(nblk,),
        in_specs=[
            pl.BlockSpec((1, EBLK, 2 * eh), lambda i: (t, i, 0)),
            pl.BlockSpec((1, EBLK, 2 * eh), lambda i: (t, i, 0)),
            pl.BlockSpec((de, EBLK), lambda i: (0, i)),
            pl.BlockSpec((de, EBLK), lambda i: (0, nblk + i)),
            pl.BlockSpec((de, eh), lambda i: (0, 0)),
            pl.BlockSpec((1, eh), lambda i: (0, 0)),
            pl.BlockSpec((eh, de), lambda i: (0, 0)),
            pl.BlockSpec((1, de), lambda i: (0, 0)),
        ],
        out_specs=pl.BlockSpec((2, EBLK, de), lambda i: (0, i, 0)),
        out_shape=jax.ShapeDtypeStruct((2, h, de), jnp.float32),
    )(gaf, gbf, eaT, eaT, w1c, b1, w2, b2)


def _make_scatter(n, de, e, ew):
    zr = n // NS
    nvb = ew // VB
    nsc = VB // SCH
    nch = ew // SCH
    mesh = plsc.VectorSubcoreMesh(core_axis_name="c", subcore_axis_name="s")

    @functools.partial(
        pl.kernel,
        out_type=[jax.ShapeDtypeStruct((NC, n, de), jnp.float32),
                  jax.ShapeDtypeStruct((NC, n, de), jnp.float32)],
        mesh=mesh,
        compiler_params=pltpu.CompilerParams(use_tc_tiling_on_sc=False),
        scratch_types=[
            pltpu.VMEM((nch, SCH), jnp.int32),
            pltpu.VMEM((2, VB, de), jnp.float32),
            pltpu.VMEM((zr, de), jnp.float32),
            pltpu.VMEM_SHARED((n, de), jnp.float32),
            pltpu.VMEM_SHARED((n, de), jnp.float32),
            pltpu.SemaphoreType.DMA,
            pltpu.SemaphoreType.DMA,
        ],
    )
    def scatter_k(ea0, ea1, dst_idx, out0, out1,
                  didx, vbuf, zbuf, sh0, sh1, lsem, ssem):
        cid = lax.axis_index("c")
        sid = lax.axis_index("s")
        wid = sid * NC + cid
        z = jnp.zeros((16,), jnp.float32)

        def zb(i, _):
            zbuf[i, :] = z
            return 0

        lax.fori_loop(0, zr, zb, 0)
        pltpu.sync_copy(zbuf, sh0.at[pl.ds(sid * zr, zr)])
        pltpu.sync_copy(zbuf, sh1.at[pl.ds(sid * zr, zr)])
        plsc.subcore_barrier()

        for t in range(2):
            eap = (ea0, ea1)[t]
            sh = (sh0, sh1)[t]
            pltpu.sync_copy(dst_idx.at[t].at[wid], didx)
            for b in range(2):
                pltpu.async_copy(
                    eap.at[pl.ds(wid * ew + b * VB, VB)], vbuf.at[b], lsem)
            for c in range(nvb):
                b = c % 2
                pltpu.make_async_copy(
                    eap.at[pl.ds(wid * ew, VB)], vbuf.at[b], lsem).wait()

                def sfire(j, _, sh=sh, c=c, b=b):
                    pltpu.async_copy(vbuf.at[b].at[pl.ds(j * SCH, SCH)],
                                     sh.at[didx.at[c * nsc + j]], ssem,
                                     add=True)
                    return 0

                def sdrain(j, _, sh=sh, b=b):
                    pltpu.make_async_copy(
                        vbuf.at[b].at[pl.ds(0, SCH)],
                        sh.at[pl.ds(0, SCH)], ssem).wait()
                    return 0

                lax.fori_loop(0, nsc, sfire, 0)
                lax.fori_loop(0, nsc, sdrain, 0)
                if c + 2 < nvb:
                    pltpu.async_copy(
                        eap.at[pl.ds(wid * ew + (c + 2) * VB, VB)],
                        vbuf.at[b], lsem)
        plsc.subcore_barrier()

        for t in range(2):
            sh = (sh0, sh1)[t]
            outp = (out0, out1)[t]
            pltpu.sync_copy(sh.at[pl.ds(sid * zr, zr)], zbuf)
            pltpu.sync_copy(zbuf, outp.at[cid].at[pl.ds(sid * zr, zr)])

    return scatter_k


def _node_body(df, de, a0_ref, a1_ref, x_ref, wn1_ref, bn1_ref, wn2_ref,
               bn2_ref, o_ref):
    agg0 = a0_ref[0] + a0_ref[1]
    agg1 = a1_ref[0] + a1_ref[1]
    hid = (jnp.dot(x_ref[...], wn1_ref[0:df],
                   preferred_element_type=jnp.float32)
           + jnp.dot(agg0, wn1_ref[df:df + de],
                     preferred_element_type=jnp.float32)
           + jnp.dot(agg1, wn1_ref[df + de:df + 2 * de],
                     preferred_element_type=jnp.float32)
           + bn1_ref[...])
    hid = jnp.maximum(hid, 0.0)
    o_ref[...] = jnp.dot(hid, wn2_ref[...],
                         preferred_element_type=jnp.float32) + bn2_ref[...]


def _node_mlp(x, a0, a1, wn1, bn1, wn2, bn2):
    n, df = x.shape
    de = a0.shape[-1]
    nh = wn1.shape[-1]
    nblk = 5
    blk = n // nblk
    return pl.pallas_call(
        functools.partial(_node_body, df, de),
        grid=(nblk,),
        in_specs=[
            pl.BlockSpec((NC, blk, de), lambda i: (0, i, 0)),
            pl.BlockSpec((NC, blk, de), lambda i: (0, i, 0)),
            pl.BlockSpec((blk, df), lambda i: (i, 0)),
            pl.BlockSpec((df + 2 * de, nh), lambda i: (0, 0)),
            pl.BlockSpec((1, nh), lambda i: (0, 0)),
            pl.BlockSpec((nh, df), lambda i: (0, 0)),
            pl.BlockSpec((1, df), lambda i: (0, 0)),
        ],
        out_specs=pl.BlockSpec((blk, df), lambda i: (i, 0)),
        out_shape=jax.ShapeDtypeStruct((n, df), jnp.float32),
    )(a0, a1, x, wn1, bn1, wn2, bn2)


def kernel(x, edge_index_0, edge_index_1, edge_attr_0, edge_attr_1,
           We1_0, be1_0, We2_0, be2_0, We1_1, be1_1, We2_1, be2_1,
           Wn1, bn1, Wn2, bn2):
    n, df = x.shape
    e = edge_index_0.shape[1]
    de = edge_attr_0.shape[-1]
    eh = We1_0.shape[-1]
    ew = e // NW
    h = e // 2

    wstack = jnp.stack([We1_0[:df], We1_0[df:2 * df],
                        We1_1[:df], We1_1[df:2 * df]])
    tab = _proj(x, wstack)
    pa0, pb0, pa1, pb1 = tab[0], tab[1], tab[2], tab[3]

    dst_idx = jnp.stack([edge_index_0[1].reshape(NW, ew // SCH, SCH),
                         edge_index_1[1].reshape(NW, ew // SCH, SCH)])

    ga, gb = _make_gather(n, eh, h, ew)(pa0, pb0, pa1, pb1,
                                        edge_index_0, edge_index_1)

    ea0T = jnp.swapaxes(edge_attr_0, 0, 1)
    ea1T = jnp.swapaxes(edge_attr_1, 0, 1)
    w1c_0 = We1_0[2 * df:]
    w1c_1 = We1_1[2 * df:]
    eaop0 = _edge_mlp(0, ga, gb, ea0T, w1c_0, be1_0.reshape(1, -1),
                      We2_0, be2_0.reshape(1, -1), h)
    eaop1 = _edge_mlp(1, ga, gb, ea1T, w1c_1, be1_1.reshape(1, -1),
                      We2_1, be2_1.reshape(1, -1), h)
    ea0 = eaop0.reshape(e, de)
    ea1 = eaop1.reshape(e, de)

    a0, a1 = _make_scatter(n, de, e, ew)(ea0, ea1, dst_idx)

    x_new = _node_mlp(x, a0, a1, Wn1, bn1.reshape(1, -1),
                      Wn2, bn2.reshape(1, -1))
    return (x_new, ea0, ea1)


# speedup vs baseline: 1.5466x; 1.5466x over previous
import functools

import jax
import jax.numpy as jnp
from jax import lax
from jax.experimental import pallas as pl
from jax.experimental.pallas import tpu as pltpu
from jax.experimental.pallas import tpu_sc as plsc

NC = 2
NS = 16
NW = NC * NS
GCH = 40
RB = 10
LAG = 5
SCH = 125
VB = 1250
EBLK = 1280


def _proj_body(x_ref, w_ref, o_ref):
    o_ref[0] = jnp.dot(x_ref[...], w_ref[0],
                       preferred_element_type=jnp.float32)


def _proj(x, wstack):
    n, df = x.shape
    eh = wstack.shape[-1]
    return pl.pallas_call(
        _proj_body,
        grid=(4,),
        in_specs=[
            pl.BlockSpec((n, df), lambda i: (0, 0)),
            pl.BlockSpec((1, df, eh), lambda i: (i, 0, 0)),
        ],
        out_specs=pl.BlockSpec((1, n, eh), lambda i: (i, 0, 0)),
        out_shape=jax.ShapeDtypeStruct((4, n, eh), jnp.float32),
    )(x, wstack)


def _make_gather(n, eh, h, ew):
    ngc = ew // GCH
    assert ngc % RB == 0 and RB == 2 * LAG
    nrounds = ngc // RB
    mesh = plsc.VectorSubcoreMesh(core_axis_name="c", subcore_axis_name="s")

    @functools.partial(
        pl.kernel,
        out_type=[jax.ShapeDtypeStruct((2, h, 2 * eh), jnp.float32),
                  jax.ShapeDtypeStruct((2, h, 2 * eh), jnp.float32)],
        mesh=mesh,
        compiler_params=pltpu.CompilerParams(use_tc_tiling_on_sc=False),
        scratch_types=[
            pltpu.VMEM((ew,), jnp.int32),
            pltpu.VMEM((ew,), jnp.int32),
            pltpu.VMEM((RB, GCH, eh), jnp.float32),
            pltpu.VMEM((RB, GCH, eh), jnp.float32),
            pltpu.SemaphoreType.DMA,
            pltpu.SemaphoreType.DMA,
        ],
    )
    def gather_k(pa0, pb0, pa1, pb1, ei0, ei1, ga, gb,
                 sidx, didx, bufa, bufb, gsem, ssem):
        cid = lax.axis_index("c")
        sid = lax.axis_index("s")
        wid = sid * NC + cid
        half = wid // NS
        rb0 = (wid % NS) * ew
        col0 = half * eh
        for t in range(2):
            pa = (pa0, pa1)[t]
            pb = (pb0, pb1)[t]
            ei = (ei0, ei1)[t]
            pltpu.sync_copy(ei.at[0].at[pl.ds(wid * ew, ew)], sidx)
            pltpu.sync_copy(ei.at[1].at[pl.ds(wid * ew, ew)], didx)
            for b in range(LAG):
                pltpu.async_copy(
                    pa.at[sidx.at[pl.ds(b * GCH, GCH)]], bufa.at[b], gsem)
                pltpu.async_copy(
                    pb.at[didx.at[pl.ds(b * GCH, GCH)]], bufb.at[b], gsem)

            def rnd(r, _, pa=pa, pb=pb, t=t):
                for b in range(RB):
                    c = r * RB + b
                    bn = (b + LAG) % RB
                    pltpu.make_async_copy(
                        pa.at[pl.ds(0, GCH)], bufa.at[b], gsem).wait()
                    pltpu.make_async_copy(
                        pb.at[pl.ds(0, GCH)], bufb.at[b], gsem).wait()
                    rb = rb0 + c * GCH
                    pltpu.async_copy(
                        bufa.at[b],
                        ga.at[t, pl.ds(rb, GCH), pl.ds(col0, eh)], ssem)
                    pltpu.async_copy(
                        bufb.at[b],
                        gb.at[t, pl.ds(rb, GCH), pl.ds(col0, eh)], ssem)

                    @pl.when(c >= LAG)
                    def _():
                        pltpu.make_async_copy(
                            bufa.at[bn],
                            ga.at[t, pl.ds(rb0, GCH), pl.ds(col0, eh)],
                            ssem).wait()
                        pltpu.make_async_copy(
                            bufb.at[bn],
                            gb.at[t, pl.ds(rb0, GCH), pl.ds(col0, eh)],
                            ssem).wait()

                    @pl.when(c + LAG < ngc)
                    def _():
                        pltpu.async_copy(
                            pa.at[sidx.at[pl.ds((c + LAG) * GCH, GCH)]],
                            bufa.at[bn], gsem)
                        pltpu.async_copy(
                            pb.at[didx.at[pl.ds((c + LAG) * GCH, GCH)]],
                            bufb.at[bn], gsem)
                return 0

            lax.fori_loop(0, nrounds, rnd, 0)
            for _ in range(LAG):
                pltpu.make_async_copy(
                    bufa.at[0],
                    ga.at[t, pl.ds(rb0, GCH), pl.ds(col0, eh)], ssem).wait()
                pltpu.make_async_copy(
                    bufb.at[0],
                    gb.at[t, pl.ds(rb0, GCH), pl.ds(col0, eh)], ssem).wait()

    return gather_k


def _edge_mlp_body(eh, de, ga_ref, gb_ref, eal_ref, ear_ref, w1c_ref,
                   b1_ref, w2_ref, b2_ref, o_ref):
    g2 = ga_ref[0] + gb_ref[0]
    for side, ea_ref in ((0, eal_ref), (1, ear_ref)):
        g = g2[:, side * eh:(side + 1) * eh]
        contrib = lax.dot_general(ea_ref[...], w1c_ref[...],
                                  (((0,), (0,)), ((), ())),
                                  preferred_element_type=jnp.float32)
        pre = g + contrib + b1_ref[...]
        hid = jnp.maximum(pre, 0.0)
        o_ref[side] = jnp.dot(hid, w2_ref[...],
                              preferred_element_type=jnp.float32) \
            + b2_ref[...]


def _edge_mlp(t, gaf, gbf, eaT, w1c, b1, w2, b2, h):
    nblk = h // EBLK
    eh = w1c.shape[-1]
    de = eaT.shape[0]
    return pl.pallas_call(
        functools.partial(_edge_mlp_body, eh, de),
        grid=(nblk,),
        in_specs=[
            pl.BlockSpec((1, EBLK, 2 * eh), lambda i: (t, i, 0)),
            pl.BlockSpec((1, EBLK, 2 * eh), lambda i: (t, i, 0)),
            pl.BlockSpec((de, EBLK), lambda i: (0, i)),
            pl.BlockSpec((de, EBLK), lambda i: (0, nblk + i)),
            pl.BlockSpec((de, eh), lambda i: (0, 0)),
            pl.BlockSpec((1, eh), lambda i: (0, 0)),
            pl.BlockSpec((eh, de), lambda i: (0, 0)),
            pl.BlockSpec((1, de), lambda i: (0, 0)),
        ],
        out_specs=pl.BlockSpec((2, EBLK, de), lambda i: (0, i, 0)),
        out_shape=jax.ShapeDtypeStruct((2, h, de), jnp.float32),
    )(gaf, gbf, eaT, eaT, w1c, b1, w2, b2)


def _make_scatter(n, de, e, ew):
    zr = n // NS
    nvb = ew // VB
    nsc = VB // SCH
    nch = ew // SCH
    mesh = plsc.VectorSubcoreMesh(core_axis_name="c", subcore_axis_name="s")

    @functools.partial(
        pl.kernel,
        out_type=[jax.ShapeDtypeStruct((NC, n, de), jnp.float32),
                  jax.ShapeDtypeStruct((NC, n, de), jnp.float32)],
        mesh=mesh,
        compiler_params=pltpu.CompilerParams(use_tc_tiling_on_sc=False),
        scratch_types=[
            pltpu.VMEM((nch, SCH), jnp.int32),
            pltpu.VMEM((2, VB, de), jnp.float32),
            pltpu.VMEM((zr, de), jnp.float32),
            pltpu.VMEM_SHARED((n, de), jnp.float32),
            pltpu.VMEM_SHARED((n, de), jnp.float32),
            pltpu.SemaphoreType.DMA,
            pltpu.SemaphoreType.DMA,
        ],
    )
    def scatter_k(ea0, ea1, dst_idx, out0, out1,
                  didx, vbuf, zbuf, sh0, sh1, lsem, ssem):
        cid = lax.axis_index("c")
        sid = lax.axis_index("s")
        wid = sid * NC + cid
        z = jnp.zeros((16,), jnp.float32)

        def zb(i, _):
            zbuf[i, :] = z
            return 0

        lax.fori_loop(0, zr, zb, 0)
        pltpu.sync_copy(zbuf, sh0.at[pl.ds(sid * zr, zr)])
        pltpu.sync_copy(zbuf, sh1.at[pl.ds(sid * zr, zr)])
        plsc.subcore_barrier()

        for t in range(2):
            eap = (ea0, ea1)[t]
            sh = (sh0, sh1)[t]
            pltpu.sync_copy(dst_idx.at[t].at[wid], didx)
            for b in range(2):
                pltpu.async_copy(
                    eap.at[pl.ds(wid * ew + b * VB, VB)], vbuf.at[b], lsem)
            for c in range(nvb):
                b = c % 2
                pltpu.make_async_copy(
                    eap.at[pl.ds(wid * ew, VB)], vbuf.at[b], lsem).wait()

                def sfire(j, _, sh=sh, c=c, b=b):
                    pltpu.async_copy(vbuf.at[b].at[pl.ds(j * SCH, SCH)],
                                     sh.at[didx.at[c * nsc + j]], ssem,
                                     add=True)
                    return 0

                def sdrain(j, _, sh=sh, b=b):
                    pltpu.make_async_copy(
                        vbuf.at[b].at[pl.ds(0, SCH)],
                        sh.at[pl.ds(0, SCH)], ssem).wait()
                    return 0

                lax.fori_loop(0, nsc, sfire, 0)
                lax.fori_loop(0, nsc, sdrain, 0)
                if c + 2 < nvb:
                    pltpu.async_copy(
                        eap.at[pl.ds(wid * ew + (c + 2) * VB, VB)],
                        vbuf.at[b], lsem)
        plsc.subcore_barrier()

        for t in range(2):
            sh = (sh0, sh1)[t]
            outp = (out0, out1)[t]
            pltpu.sync_copy(sh.at[pl.ds(sid * zr, zr)], zbuf)
            pltpu.sync_copy(zbuf, outp.at[cid].at[pl.ds(sid * zr, zr)])

    return scatter_k


def _node_body(df, de, a0_ref, a1_ref, x_ref, wn1_ref, bn1_ref, wn2_ref,
               bn2_ref, o_ref):
    agg0 = a0_ref[0] + a0_ref[1]
    agg1 = a1_ref[0] + a1_ref[1]
    hid = (jnp.dot(x_ref[...], wn1_ref[0:df],
                   preferred_element_type=jnp.float32)
           + jnp.dot(agg0, wn1_ref[df:df + de],
                     preferred_element_type=jnp.float32)
           + jnp.dot(agg1, wn1_ref[df + de:df + 2 * de],
                     preferred_element_type=jnp.float32)
           + bn1_ref[...])
    hid = jnp.maximum(hid, 0.0)
    o_ref[...] = jnp.dot(hid, wn2_ref[...],
                         preferred_element_type=jnp.float32) + bn2_ref[...]


def _node_mlp(x, a0, a1, wn1, bn1, wn2, bn2):
    n, df = x.shape
    de = a0.shape[-1]
    nh = wn1.shape[-1]
    nblk = 5
    blk = n // nblk
    return pl.pallas_call(
        functools.partial(_node_body, df, de),
        grid=(nblk,),
        in_specs=[
            pl.BlockSpec((NC, blk, de), lambda i: (0, i, 0)),
            pl.BlockSpec((NC, blk, de), lambda i: (0, i, 0)),
            pl.BlockSpec((blk, df), lambda i: (i, 0)),
            pl.BlockSpec((df + 2 * de, nh), lambda i: (0, 0)),
            pl.BlockSpec((1, nh), lambda i: (0, 0)),
            pl.BlockSpec((nh, df), lambda i: (0, 0)),
            pl.BlockSpec((1, df), lambda i: (0, 0)),
        ],
        out_specs=pl.BlockSpec((blk, df), lambda i: (i, 0)),
        out_shape=jax.ShapeDtypeStruct((n, df), jnp.float32),
    )(a0, a1, x, wn1, bn1, wn2, bn2)


def kernel(x, edge_index_0, edge_index_1, edge_attr_0, edge_attr_1,
           We1_0, be1_0, We2_0, be2_0, We1_1, be1_1, We2_1, be2_1,
           Wn1, bn1, Wn2, bn2):
    n, df = x.shape
    e = edge_index_0.shape[1]
    de = edge_attr_0.shape[-1]
    eh = We1_0.shape[-1]
    ew = e // NW
    h = e // 2

    wstack = jnp.stack([We1_0[:df], We1_0[df:2 * df],
                        We1_1[:df], We1_1[df:2 * df]])
    tab = _proj(x, wstack)
    pa0, pb0, pa1, pb1 = tab[0], tab[1], tab[2], tab[3]

    dst_idx = jnp.stack([edge_index_0[1].reshape(NW, ew // SCH, SCH),
                         edge_index_1[1].reshape(NW, ew // SCH, SCH)])

    ga, gb = _make_gather(n, eh, h, ew)(pa0, pb0, pa1, pb1,
                                        edge_index_0, edge_index_1)

    ea0T = jnp.swapaxes(edge_attr_0, 0, 1)
    ea1T = jnp.swapaxes(edge_attr_1, 0, 1)
    w1c_0 = We1_0[2 * df:]
    w1c_1 = We1_1[2 * df:]
    eaop0 = _edge_mlp(0, ga, gb, ea0T, w1c_0, be1_0.reshape(1, -1),
                      We2_0, be2_0.reshape(1, -1), h)
    eaop1 = _edge_mlp(1, ga, gb, ea1T, w1c_1, be1_1.reshape(1, -1),
                      We2_1, be2_1.reshape(1, -1), h)
    ea0 = eaop0.reshape(e, de)
    ea1 = eaop1.reshape(e, de)

    a0, a1 = _make_scatter(n, de, e, ew)(ea0, ea1, dst_idx)

    x_new = _node_mlp(x, a0, a1, Wn1, bn1.reshape(1, -1),
                      Wn2, bn2.reshape(1, -1))
    return (x_new, ea0, ea1)
